# Initial kernel scaffold; baseline (speedup 1.0000x reference)
import functools

import jax
import jax.numpy as jnp
from jax import lax
from jax.experimental import pallas as pl
from jax.experimental.pallas import tpu as pltpu
from jax.experimental.pallas import tpu_sc as plsc

N = 10000
E = 320000
D_IN = 128
H = 64
NC = 2
NS = 16
NW = NC * NS
CHUNK = 128
CPW = 80
EPW = CPW * CHUNK
E_PAD = NW * EPW
NP = 10240
RPS = NP // NS
BLK = 1280
F32 = jnp.float32

_mesh = plsc.VectorSubcoreMesh(core_axis_name="c", subcore_axis_name="s")


@functools.partial(
    pl.kernel,
    out_type=[jax.ShapeDtypeStruct((NC, NP, H), F32),
              jax.ShapeDtypeStruct((NC, NP), F32)],
    mesh=_mesh,
    scratch_types=[
        pltpu.VMEM((CPW, CHUNK), jnp.int32),
        pltpu.VMEM((CPW, CHUNK), jnp.int32),
        pltpu.VMEM((CHUNK, H), F32),
        pltpu.VMEM((CHUNK,), F32),
        pltpu.VMEM_SHARED((NP, H), F32),
        pltpu.VMEM_SHARED((NP,), F32),
    ],
)
def _seg64_deg(y, colr, rowr, z2, z1, on, accout, degout,
               col_v, row_v, gbuf, ones_v, acc, dacc):
    ci = lax.axis_index("c")
    si = lax.axis_index("s")
    wid = ci * NS + si
    pltpu.sync_copy(z2, acc.at[pl.ds(si * RPS, RPS)])
    pltpu.sync_copy(z1, dacc.at[pl.ds(si * RPS, RPS)])
    pltpu.sync_copy(on, ones_v)
    pltpu.sync_copy(colr.at[wid], col_v)
    pltpu.sync_copy(rowr.at[wid], row_v)
    plsc.subcore_barrier()

    def body(j, carry):
        pltpu.sync_copy(y.at[col_v.at[j]], gbuf)
        pltpu.sync_copy(gbuf, acc.at[row_v.at[j]], add=True)
        pltpu.sync_copy(ones_v, dacc.at[row_v.at[j]], add=True)
        return carry

    lax.fori_loop(0, CPW, body, 0)
    plsc.subcore_barrier()
    pltpu.sync_copy(acc.at[pl.ds(si * RPS, RPS)],
                    accout.at[ci, pl.ds(si * RPS, RPS)])
    pltpu.sync_copy(dacc.at[pl.ds(si * RPS, RPS)],
                    degout.at[ci, pl.ds(si * RPS, RPS)])


@functools.partial(
    pl.kernel,
    out_type=jax.ShapeDtypeStruct((NC, NP, H), F32),
    mesh=_mesh,
    scratch_types=[
        pltpu.VMEM((CPW, CHUNK), jnp.int32),
        pltpu.VMEM((CPW, CHUNK), jnp.int32),
        pltpu.VMEM((CHUNK, H), F32),
        pltpu.VMEM_SHARED((NP, H), F32),
    ],
)
def _seg64(y, colr, rowr, z2, accout, col_v, row_v, gbuf, acc):
    ci = lax.axis_index("c")
    si = lax.axis_index("s")
    wid = ci * NS + si
    pltpu.sync_copy(z2, acc.at[pl.ds(si * RPS, RPS)])
    pltpu.sync_copy(colr.at[wid], col_v)
    pltpu.sync_copy(rowr.at[wid], row_v)
    plsc.subcore_barrier()

    def body(j, carry):
        pltpu.sync_copy(y.at[col_v.at[j]], gbuf)
        pltpu.sync_copy(gbuf, acc.at[row_v.at[j]], add=True)
        return carry

    lax.fori_loop(0, CPW, body, 0)
    plsc.subcore_barrier()
    pltpu.sync_copy(acc.at[pl.ds(si * RPS, RPS)],
                    accout.at[ci, pl.ds(si * RPS, RPS)])


@functools.partial(
    pl.kernel,
    out_type=jax.ShapeDtypeStruct((NC, NP), F32),
    mesh=_mesh,
    scratch_types=[
        pltpu.VMEM((CPW, CHUNK), jnp.int32),
        pltpu.VMEM((CPW, CHUNK), jnp.int32),
        pltpu.VMEM((CHUNK,), F32),
        pltpu.VMEM_SHARED((NP,), F32),
    ],
)
def _seg1(y, colr, rowr, z1, accout, col_v, row_v, gbuf, acc):
    ci = lax.axis_index("c")
    si = lax.axis_index("s")
    wid = ci * NS + si
    pltpu.sync_copy(z1, acc.at[pl.ds(si * RPS, RPS)])
    pltpu.sync_copy(colr.at[wid], col_v)
    pltpu.sync_copy(rowr.at[wid], row_v)
    plsc.subcore_barrier()

    def body(j, carry):
        pltpu.sync_copy(y.at[col_v.at[j]], gbuf)
        pltpu.sync_copy(gbuf, acc.at[row_v.at[j]], add=True)
        return carry

    lax.fori_loop(0, CPW, body, 0)
    plsc.subcore_barrier()
    pltpu.sync_copy(acc.at[pl.ds(si * RPS, RPS)],
                    accout.at[ci, pl.ds(si * RPS, RPS)])


def _row_spec(width):
    return pl.BlockSpec((BLK, width), lambda i: (i, 0))


def _fixed_spec(shape):
    nd = len(shape)
    return pl.BlockSpec(shape, lambda i: (0,) * nd)


def _pair_spec(width):
    return pl.BlockSpec((NC, BLK, width), lambda i: (0, i, 0))


def _tc0(x, wl, wr):
    def body(x_ref, wl_ref, wr_ref, y_ref, z_ref):
        xb = x_ref[...]
        y_ref[...] = jnp.dot(xb, wl_ref[...], preferred_element_type=F32)
        z_ref[...] = jnp.dot(xb, wr_ref[...], preferred_element_type=F32)

    return pl.pallas_call(
        body,
        grid=(NP // BLK,),
        in_specs=[_row_spec(D_IN), _fixed_spec((D_IN, H)), _fixed_spec((D_IN, H))],
        out_specs=[_row_spec(H), _row_spec(H)],
        out_shape=[jax.ShapeDtypeStruct((NP, H), F32)] * 2,
    )(x, wl, wr)


def _tc1(accp, degp, z, bl, wln, wrn):
    def body(a_ref, d_ref, z_ref, bl_ref, wl_ref, wr_ref,
             rdeg_ref, y_ref, z2_ref):
        deg = d_ref[0] + d_ref[1]
        rdeg = 1.0 / jnp.maximum(deg, 1.0)
        h = jnp.maximum((a_ref[0] + a_ref[1]) * rdeg + z_ref[...] + bl_ref[...],
                        0.0)
        rdeg_ref[...] = rdeg
        y_ref[...] = jnp.dot(h, wl_ref[...], preferred_element_type=F32)
        z2_ref[...] = jnp.dot(h, wr_ref[...], preferred_element_type=F32)

    return pl.pallas_call(
        body,
        grid=(NP // BLK,),
        in_specs=[_pair_spec(H), _pair_spec(1), _row_spec(H),
                  _fixed_spec((1, H)), _fixed_spec((H, H)), _fixed_spec((H, H))],
        out_specs=[_row_spec(1), _row_spec(H), _row_spec(H)],
        out_shape=[jax.ShapeDtypeStruct((NP, 1), F32),
                   jax.ShapeDtypeStruct((NP, H), F32),
                   jax.ShapeDtypeStruct((NP, H), F32)],
    )(accp, degp, z, bl, wln, wrn)


def _tcmid(accp, rdeg, z, bl, wln, wrn):
    def body(a_ref, rd_ref, z_ref, bl_ref, wl_ref, wr_ref, y_ref, z2_ref):
        h = jnp.maximum((a_ref[0] + a_ref[1]) * rd_ref[...] + z_ref[...]
                        + bl_ref[...], 0.0)
        y_ref[...] = jnp.dot(h, wl_ref[...], preferred_element_type=F32)
        z2_ref[...] = jnp.dot(h, wr_ref[...], preferred_element_type=F32)

    return pl.pallas_call(
        body,
        grid=(NP // BLK,),
        in_specs=[_pair_spec(H), _row_spec(1), _row_spec(H),
                  _fixed_spec((1, H)), _fixed_spec((H, H)), _fixed_spec((H, H))],
        out_specs=[_row_spec(H), _row_spec(H)],
        out_shape=[jax.ShapeDtypeStruct((NP, H), F32)] * 2,
    )(accp, rdeg, z, bl, wln, wrn)


def _tc4(accp, rdeg, z, bl, whead):
    def body(a_ref, rd_ref, z_ref, bl_ref, wh_ref, p_ref):
        h = jnp.maximum((a_ref[0] + a_ref[1]) * rd_ref[...] + z_ref[...]
                        + bl_ref[...], 0.0)
        p_ref[...] = jnp.dot(h, wh_ref[...], preferred_element_type=F32)

    return pl.pallas_call(
        body,
        grid=(NP // BLK,),
        in_specs=[_pair_spec(H), _row_spec(1), _row_spec(H),
                  _fixed_spec((1, H)), _fixed_spec((H, 4))],
        out_specs=_row_spec(4),
        out_shape=jax.ShapeDtypeStruct((NP, 4), F32),
    )(accp, rdeg, z, bl, whead)


def _tc5(paccp, rdeg, p, b3, m3):
    def body(pa_ref, rd_ref, p_ref, b3_ref, m3_ref, out_ref):
        pr = (pa_ref[0] + pa_ref[1]) * rd_ref[...]
        out_ref[...] = p_ref[:, 1:4] + b3_ref[...] + pr * m3_ref[...]

    return pl.pallas_call(
        body,
        grid=(NP // BLK,),
        in_specs=[_pair_spec(1), _row_spec(1), _row_spec(4),
                  _fixed_spec((1, 3)), _fixed_spec((1, 3))],
        out_specs=_row_spec(3),
        out_shape=jax.ShapeDtypeStruct((NP, 3), F32),
    )(paccp, rdeg, p, b3, m3)


def kernel(x, edge_index, Wl0, bl0, Wr0, Wl1, bl1, Wr1, Wl2, bl2, Wr2,
           Wl3, bl3, Wr3, Wlp, blp, Wrp, Wdn, bdn, Wv, bv):
    row = edge_index[0]
    col = edge_index[1]
    pad = E_PAD - E
    rowp = jnp.concatenate(
        [row, jnp.full((pad,), N, jnp.int32)]).reshape(NW, CPW, CHUNK)
    colp = jnp.concatenate(
        [col, jnp.zeros((pad,), jnp.int32)]).reshape(NW, CPW, CHUNK)
    xp = jnp.pad(x, ((0, NP - N), (0, 0)))
    z2 = jnp.zeros((RPS, H), F32)
    z1 = jnp.zeros((RPS,), F32)
    on = jnp.ones((CHUNK,), F32)

    y0, zz0 = _tc0(xp, Wl0, Wr0)
    acc0, deg0 = _seg64_deg(y0, colp, rowp, z2, z1, on)
    rdeg, y1, zz1 = _tc1(acc0, deg0.reshape(NC, NP, 1), zz0,
                         bl0.reshape(1, H), Wl1, Wr1)
    acc1 = _seg64(y1, colp, rowp, z2)
    y2, zz2 = _tcmid(acc1, rdeg, zz1, bl1.reshape(1, H), Wl2, Wr2)
    acc2 = _seg64(y2, colp, rowp, z2)
    y3, zz3 = _tcmid(acc2, rdeg, zz2, bl2.reshape(1, H), Wl3, Wr3)
    acc3 = _seg64(y3, colp, rowp, z2)
    whead = jnp.concatenate([Wlp, Wrp, Wdn, Wv], axis=1)
    p = _tc4(acc3, rdeg, zz3, bl3.reshape(1, H), whead)
    accp = _seg1(p[:, 0], colp, rowp, z1)
    b3 = jnp.stack([blp[0], bdn[0], bv[0]]).reshape(1, 3)
    m3 = jnp.array([[1.0, 0.0, 0.0]], F32)
    out = _tc5(accp.reshape(NC, NP, 1), rdeg, p, b3, m3)
    return out[:N]


# speedup vs baseline: 5.2128x; 5.2128x over previous
import functools

import jax
import jax.numpy as jnp
from jax import lax
from jax.experimental import pallas as pl
from jax.experimental.pallas import tpu as pltpu
from jax.experimental.pallas import tpu_sc as plsc

N = 10000
E = 320000
D_IN = 128
H = 64
NC = 2
NS = 16
NW = NC * NS
CHUNK = 128
CPW = 80
EPW = CPW * CHUNK
E_PAD = NW * EPW
NP = 10240
RPS = NP // NS
BLK = 1280
F32 = jnp.float32

@functools.lru_cache(maxsize=None)
def _sc_kernels():
    mesh = plsc.VectorSubcoreMesh(core_axis_name="c", subcore_axis_name="s",
                                  num_cores=NC, num_subcores=NS)

    @functools.partial(
        pl.kernel,
        out_type=[jax.ShapeDtypeStruct((NC, NP, H), F32),
                  jax.ShapeDtypeStruct((NC, NP), F32)],
        mesh=mesh,
        compiler_params=pltpu.CompilerParams(use_tc_tiling_on_sc=False),
        scratch_types=[
            pltpu.VMEM((CPW, CHUNK), jnp.int32),
            pltpu.VMEM((CPW, CHUNK), jnp.int32),
            pltpu.VMEM((CHUNK, H), F32),
            pltpu.VMEM((CHUNK,), F32),
            pltpu.VMEM_SHARED((NP, H), F32),
            pltpu.VMEM_SHARED((NP,), F32),
        ],
    )
    def seg64_deg(y, colr, rowr, z2, z1, on, accout, degout,
                  col_v, row_v, gbuf, ones_v, acc, dacc):
        ci = lax.axis_index("c")
        si = lax.axis_index("s")
        wid = ci * NS + si
        pltpu.sync_copy(z2, acc.at[pl.ds(si * RPS, RPS)])
        pltpu.sync_copy(z1, dacc.at[pl.ds(si * RPS, RPS)])
        pltpu.sync_copy(on, ones_v)
        pltpu.sync_copy(colr.at[wid], col_v)
        pltpu.sync_copy(rowr.at[wid], row_v)
        plsc.subcore_barrier()

        def body(j, carry):
            pltpu.sync_copy(y.at[col_v.at[j]], gbuf)
            pltpu.sync_copy(gbuf, acc.at[row_v.at[j]], add=True)
            pltpu.sync_copy(ones_v, dacc.at[row_v.at[j]], add=True)
            return carry

        lax.fori_loop(0, CPW, body, 0)
        plsc.subcore_barrier()
        pltpu.sync_copy(acc.at[pl.ds(si * RPS, RPS)],
                        accout.at[ci, pl.ds(si * RPS, RPS)])
        pltpu.sync_copy(dacc.at[pl.ds(si * RPS, RPS)],
                        degout.at[ci, pl.ds(si * RPS, RPS)])

    @functools.partial(
        pl.kernel,
        out_type=jax.ShapeDtypeStruct((NC, NP, H), F32),
        mesh=mesh,
        compiler_params=pltpu.CompilerParams(use_tc_tiling_on_sc=False),
        scratch_types=[
            pltpu.VMEM((CPW, CHUNK), jnp.int32),
            pltpu.VMEM((CPW, CHUNK), jnp.int32),
            pltpu.VMEM((CHUNK, H), F32),
            pltpu.VMEM_SHARED((NP, H), F32),
        ],
    )
    def seg64(y, colr, rowr, z2, accout, col_v, row_v, gbuf, acc):
        ci = lax.axis_index("c")
        si = lax.axis_index("s")
        wid = ci * NS + si
        pltpu.sync_copy(z2, acc.at[pl.ds(si * RPS, RPS)])
        pltpu.sync_copy(colr.at[wid], col_v)
        pltpu.sync_copy(rowr.at[wid], row_v)
        plsc.subcore_barrier()

        def body(j, carry):
            pltpu.sync_copy(y.at[col_v.at[j]], gbuf)
            pltpu.sync_copy(gbuf, acc.at[row_v.at[j]], add=True)
            return carry

        lax.fori_loop(0, CPW, body, 0)
        plsc.subcore_barrier()
        pltpu.sync_copy(acc.at[pl.ds(si * RPS, RPS)],
                        accout.at[ci, pl.ds(si * RPS, RPS)])

    @functools.partial(
        pl.kernel,
        out_type=jax.ShapeDtypeStruct((NC, NP), F32),
        mesh=mesh,
        compiler_params=pltpu.CompilerParams(use_tc_tiling_on_sc=False),
        scratch_types=[
            pltpu.VMEM((CPW, CHUNK), jnp.int32),
            pltpu.VMEM((CPW, CHUNK), jnp.int32),
            pltpu.VMEM((CHUNK,), F32),
            pltpu.VMEM_SHARED((NP,), F32),
        ],
    )
    def seg1(y, colr, rowr, z1, accout, col_v, row_v, gbuf, acc):
        ci = lax.axis_index("c")
        si = lax.axis_index("s")
        wid = ci * NS + si
        pltpu.sync_copy(z1, acc.at[pl.ds(si * RPS, RPS)])
        pltpu.sync_copy(colr.at[wid], col_v)
        pltpu.sync_copy(rowr.at[wid], row_v)
        plsc.subcore_barrier()

        def body(j, carry):
            pltpu.sync_copy(y.at[col_v.at[j]], gbuf)
            pltpu.sync_copy(gbuf, acc.at[row_v.at[j]], add=True)
            return carry

        lax.fori_loop(0, CPW, body, 0)
        plsc.subcore_barrier()
        pltpu.sync_copy(acc.at[pl.ds(si * RPS, RPS)],
                        accout.at[ci, pl.ds(si * RPS, RPS)])

    return seg64_deg, seg64, seg1


def _row_spec(width):
    return pl.BlockSpec((BLK, width), lambda i: (i, 0))


def _fixed_spec(shape):
    nd = len(shape)
    return pl.BlockSpec(shape, lambda i: (0,) * nd)


def _pair_spec(width):
    return pl.BlockSpec((NC, BLK, width), lambda i: (0, i, 0))


def _tc0(x, wl, wr):
    def body(x_ref, wl_ref, wr_ref, y_ref, z_ref):
        xb = x_ref[...]
        y_ref[...] = jnp.dot(xb, wl_ref[...], preferred_element_type=F32)
        z_ref[...] = jnp.dot(xb, wr_ref[...], preferred_element_type=F32)

    return pl.pallas_call(
        body,
        grid=(NP // BLK,),
        in_specs=[_row_spec(D_IN), _fixed_spec((D_IN, H)), _fixed_spec((D_IN, H))],
        out_specs=[_row_spec(H), _row_spec(H)],
        out_shape=[jax.ShapeDtypeStruct((NP, H), F32)] * 2,
    )(x, wl, wr)


def _tc1(accp, degp, z, bl, wln, wrn):
    def body(a_ref, d_ref, z_ref, bl_ref, wl_ref, wr_ref,
             rdeg_ref, y_ref, z2_ref):
        deg = d_ref[0] + d_ref[1]
        rdeg = 1.0 / jnp.maximum(deg, 1.0)
        h = jnp.maximum((a_ref[0] + a_ref[1]) * rdeg + z_ref[...] + bl_ref[...],
                        0.0)
        rdeg_ref[...] = rdeg
        y_ref[...] = jnp.dot(h, wl_ref[...], preferred_element_type=F32)
        z2_ref[...] = jnp.dot(h, wr_ref[...], preferred_element_type=F32)

    return pl.pallas_call(
        body,
        grid=(NP // BLK,),
        in_specs=[_pair_spec(H), _pair_spec(1), _row_spec(H),
                  _fixed_spec((1, H)), _fixed_spec((H, H)), _fixed_spec((H, H))],
        out_specs=[_row_spec(1), _row_spec(H), _row_spec(H)],
        out_shape=[jax.ShapeDtypeStruct((NP, 1), F32),
                   jax.ShapeDtypeStruct((NP, H), F32),
                   jax.ShapeDtypeStruct((NP, H), F32)],
    )(accp, degp, z, bl, wln, wrn)


def _tcmid(accp, rdeg, z, bl, wln, wrn):
    def body(a_ref, rd_ref, z_ref, bl_ref, wl_ref, wr_ref, y_ref, z2_ref):
        h = jnp.maximum((a_ref[0] + a_ref[1]) * rd_ref[...] + z_ref[...]
                        + bl_ref[...], 0.0)
        y_ref[...] = jnp.dot(h, wl_ref[...], preferred_element_type=F32)
        z2_ref[...] = jnp.dot(h, wr_ref[...], preferred_element_type=F32)

    return pl.pallas_call(
        body,
        grid=(NP // BLK,),
        in_specs=[_pair_spec(H), _row_spec(1), _row_spec(H),
                  _fixed_spec((1, H)), _fixed_spec((H, H)), _fixed_spec((H, H))],
        out_specs=[_row_spec(H), _row_spec(H)],
        out_shape=[jax.ShapeDtypeStruct((NP, H), F32)] * 2,
    )(accp, rdeg, z, bl, wln, wrn)


def _tc4(accp, rdeg, z, bl, whead):
    def body(a_ref, rd_ref, z_ref, bl_ref, wh_ref, p_ref):
        h = jnp.maximum((a_ref[0] + a_ref[1]) * rd_ref[...] + z_ref[...]
                        + bl_ref[...], 0.0)
        p_ref[...] = jnp.dot(h, wh_ref[...], preferred_element_type=F32)

    return pl.pallas_call(
        body,
        grid=(NP // BLK,),
        in_specs=[_pair_spec(H), _row_spec(1), _row_spec(H),
                  _fixed_spec((1, H)), _fixed_spec((H, 4))],
        out_specs=_row_spec(4),
        out_shape=jax.ShapeDtypeStruct((NP, 4), F32),
    )(accp, rdeg, z, bl, whead)


def _tc5(paccp, rdeg, p, b3, m3):
    def body(pa_ref, rd_ref, p_ref, b3_ref, m3_ref, out_ref):
        pr = (pa_ref[0] + pa_ref[1]) * rd_ref[...]
        out_ref[...] = p_ref[:, 1:4] + b3_ref[...] + pr * m3_ref[...]

    return pl.pallas_call(
        body,
        grid=(NP // BLK,),
        in_specs=[_pair_spec(1), _row_spec(1), _row_spec(4),
                  _fixed_spec((1, 3)), _fixed_spec((1, 3))],
        out_specs=_row_spec(3),
        out_shape=jax.ShapeDtypeStruct((NP, 3), F32),
    )(paccp, rdeg, p, b3, m3)


def kernel(x, edge_index, Wl0, bl0, Wr0, Wl1, bl1, Wr1, Wl2, bl2, Wr2,
           Wl3, bl3, Wr3, Wlp, blp, Wrp, Wdn, bdn, Wv, bv):
    row = edge_index[0]
    col = edge_index[1]
    pad = E_PAD - E
    rowp = jnp.concatenate(
        [row, jnp.full((pad,), N, jnp.int32)]).reshape(NW, CPW, CHUNK)
    colp = jnp.concatenate(
        [col, jnp.zeros((pad,), jnp.int32)]).reshape(NW, CPW, CHUNK)
    xp = jnp.pad(x, ((0, NP - N), (0, 0)))
    z2 = jnp.zeros((RPS, H), F32)
    z1 = jnp.zeros((RPS,), F32)
    on = jnp.ones((CHUNK,), F32)

    _seg64_deg, _seg64, _seg1 = _sc_kernels()

    y0, zz0 = _tc0(xp, Wl0, Wr0)
    acc0, deg0 = _seg64_deg(y0, colp, rowp, z2, z1, on)
    rdeg, y1, zz1 = _tc1(acc0, deg0.reshape(NC, NP, 1), zz0,
                         bl0.reshape(1, H), Wl1, Wr1)
    acc1 = _seg64(y1, colp, rowp, z2)
    y2, zz2 = _tcmid(acc1, rdeg, zz1, bl1.reshape(1, H), Wl2, Wr2)
    acc2 = _seg64(y2, colp, rowp, z2)
    y3, zz3 = _tcmid(acc2, rdeg, zz2, bl2.reshape(1, H), Wl3, Wr3)
    acc3 = _seg64(y3, colp, rowp, z2)
    whead = jnp.concatenate([Wlp, Wrp, Wdn, Wv], axis=1)
    p = _tc4(acc3, rdeg, zz3, bl3.reshape(1, H), whead)
    accp = _seg1(p[:, 0], colp, rowp, z1)
    b3 = jnp.stack([blp[0], bdn[0], bv[0]]).reshape(1, 3)
    m3 = jnp.array([[1.0, 0.0, 0.0]], F32)
    out = _tc5(accp.reshape(NC, NP, 1), rdeg, p, b3, m3)
    return out[:N]


# speedup vs baseline: 5.4411x; 1.0438x over previous
import functools

import jax
import jax.numpy as jnp
from jax import lax
from jax.experimental import pallas as pl
from jax.experimental.pallas import tpu as pltpu
from jax.experimental.pallas import tpu_sc as plsc

N = 10000
E = 320000
D_IN = 128
H = 64
NC = 2
NS = 16
NW = NC * NS
CHUNK = 128
CPW = 80
EPW = CPW * CHUNK
E_PAD = NW * EPW
NP = 10240
RPS = NP // NS
BLK = 1280
F32 = jnp.float32

NBUF = 8
NROUND = CPW // NBUF


def _ring(y, col_v, row_v, acc, gbuf, gsem, ssem, deg=None):
    for b in range(NBUF):
        pltpu.async_copy(y.at[col_v.at[b]], gbuf.at[b], gsem.at[b])

    def round_body(g, carry):
        base = g * NBUF
        for b in range(NBUF):
            j = base + b
            pltpu.make_async_copy(y.at[col_v.at[j]], gbuf.at[b],
                                  gsem.at[b]).wait()
            pltpu.async_copy(gbuf.at[b], acc.at[row_v.at[j]], ssem.at[b],
                             add=True)
            if deg is not None:
                ones_v, dacc, dsem = deg
                pltpu.async_copy(ones_v, dacc.at[row_v.at[j]], dsem.at[b],
                                 add=True)
        for b in range(NBUF):
            j = base + b
            jn = jnp.minimum(j + NBUF, CPW - 1)
            pltpu.make_async_copy(gbuf.at[b], acc.at[row_v.at[j]],
                                  ssem.at[b]).wait()
            if deg is not None:
                ones_v, dacc, dsem = deg
                pltpu.make_async_copy(ones_v, dacc.at[row_v.at[j]],
                                      dsem.at[b]).wait()
            pltpu.async_copy(y.at[col_v.at[jn]], gbuf.at[b], gsem.at[b])
        return carry

    lax.fori_loop(0, NROUND, round_body, 0)
    for b in range(NBUF):
        pltpu.make_async_copy(y.at[col_v.at[CPW - 1]], gbuf.at[b],
                              gsem.at[b]).wait()


@functools.lru_cache(maxsize=None)
def _sc_kernels():
    mesh = plsc.VectorSubcoreMesh(core_axis_name="c", subcore_axis_name="s",
                                  num_cores=NC, num_subcores=NS)

    @functools.partial(
        pl.kernel,
        out_type=[jax.ShapeDtypeStruct((NC, NP, H), F32),
                  jax.ShapeDtypeStruct((NC, NP), F32)],
        mesh=mesh,
        compiler_params=pltpu.CompilerParams(use_tc_tiling_on_sc=False),
        scratch_types=[
            pltpu.VMEM((CPW, CHUNK), jnp.int32),
            pltpu.VMEM((CPW, CHUNK), jnp.int32),
            pltpu.VMEM((NBUF, CHUNK, H), F32),
            pltpu.VMEM((CHUNK,), F32),
            pltpu.VMEM_SHARED((NP, H), F32),
            pltpu.VMEM_SHARED((NP,), F32),
            pltpu.SemaphoreType.DMA((NBUF,)),
            pltpu.SemaphoreType.DMA((NBUF,)),
            pltpu.SemaphoreType.DMA((NBUF,)),
        ],
    )
    def seg64_deg(y, colr, rowr, z2, z1, on, accout, degout,
                  col_v, row_v, gbuf, ones_v, acc, dacc, gsem, ssem, dsem):
        ci = lax.axis_index("c")
        si = lax.axis_index("s")
        wid = ci * NS + si
        pltpu.sync_copy(z2, acc.at[pl.ds(si * RPS, RPS)])
        pltpu.sync_copy(z1, dacc.at[pl.ds(si * RPS, RPS)])
        pltpu.sync_copy(on, ones_v)
        pltpu.sync_copy(colr.at[wid], col_v)
        pltpu.sync_copy(rowr.at[wid], row_v)
        plsc.subcore_barrier()
        _ring(y, col_v, row_v, acc, gbuf, gsem, ssem,
              deg=(ones_v, dacc, dsem))
        plsc.subcore_barrier()
        pltpu.sync_copy(acc.at[pl.ds(si * RPS, RPS)],
                        accout.at[ci, pl.ds(si * RPS, RPS)])
        pltpu.sync_copy(dacc.at[pl.ds(si * RPS, RPS)],
                        degout.at[ci, pl.ds(si * RPS, RPS)])

    @functools.partial(
        pl.kernel,
        out_type=jax.ShapeDtypeStruct((NC, NP, H), F32),
        mesh=mesh,
        compiler_params=pltpu.CompilerParams(use_tc_tiling_on_sc=False),
        scratch_types=[
            pltpu.VMEM((CPW, CHUNK), jnp.int32),
            pltpu.VMEM((CPW, CHUNK), jnp.int32),
            pltpu.VMEM((NBUF, CHUNK, H), F32),
            pltpu.VMEM_SHARED((NP, H), F32),
            pltpu.SemaphoreType.DMA((NBUF,)),
            pltpu.SemaphoreType.DMA((NBUF,)),
        ],
    )
    def seg64(y, colr, rowr, z2, accout, col_v, row_v, gbuf, acc, gsem, ssem):
        ci = lax.axis_index("c")
        si = lax.axis_index("s")
        wid = ci * NS + si
        pltpu.sync_copy(z2, acc.at[pl.ds(si * RPS, RPS)])
        pltpu.sync_copy(colr.at[wid], col_v)
        pltpu.sync_copy(rowr.at[wid], row_v)
        plsc.subcore_barrier()
        _ring(y, col_v, row_v, acc, gbuf, gsem, ssem)
        plsc.subcore_barrier()
        pltpu.sync_copy(acc.at[pl.ds(si * RPS, RPS)],
                        accout.at[ci, pl.ds(si * RPS, RPS)])

    @functools.partial(
        pl.kernel,
        out_type=jax.ShapeDtypeStruct((NC, NP), F32),
        mesh=mesh,
        compiler_params=pltpu.CompilerParams(use_tc_tiling_on_sc=False),
        scratch_types=[
            pltpu.VMEM((CPW, CHUNK), jnp.int32),
            pltpu.VMEM((CPW, CHUNK), jnp.int32),
            pltpu.VMEM((NBUF, CHUNK), F32),
            pltpu.VMEM_SHARED((NP,), F32),
            pltpu.SemaphoreType.DMA((NBUF,)),
            pltpu.SemaphoreType.DMA((NBUF,)),
        ],
    )
    def seg1(y, colr, rowr, z1, accout, col_v, row_v, gbuf, acc, gsem, ssem):
        ci = lax.axis_index("c")
        si = lax.axis_index("s")
        wid = ci * NS + si
        pltpu.sync_copy(z1, acc.at[pl.ds(si * RPS, RPS)])
        pltpu.sync_copy(colr.at[wid], col_v)
        pltpu.sync_copy(rowr.at[wid], row_v)
        plsc.subcore_barrier()
        _ring(y, col_v, row_v, acc, gbuf, gsem, ssem)
        plsc.subcore_barrier()
        pltpu.sync_copy(acc.at[pl.ds(si * RPS, RPS)],
                        accout.at[ci, pl.ds(si * RPS, RPS)])

    return seg64_deg, seg64, seg1


def _row_spec(width):
    return pl.BlockSpec((BLK, width), lambda i: (i, 0))


def _fixed_spec(shape):
    nd = len(shape)
    return pl.BlockSpec(shape, lambda i: (0,) * nd)


def _pair_spec(width):
    return pl.BlockSpec((NC, BLK, width), lambda i: (0, i, 0))


def _tc0(x, wl, wr):
    def body(x_ref, wl_ref, wr_ref, y_ref, z_ref):
        xb = x_ref[...]
        y_ref[...] = jnp.dot(xb, wl_ref[...], preferred_element_type=F32)
        z_ref[...] = jnp.dot(xb, wr_ref[...], preferred_element_type=F32)

    return pl.pallas_call(
        body,
        grid=(NP // BLK,),
        in_specs=[_row_spec(D_IN), _fixed_spec((D_IN, H)), _fixed_spec((D_IN, H))],
        out_specs=[_row_spec(H), _row_spec(H)],
        out_shape=[jax.ShapeDtypeStruct((NP, H), F32)] * 2,
    )(x, wl, wr)


def _tc1(accp, degp, z, bl, wln, wrn):
    def body(a_ref, d_ref, z_ref, bl_ref, wl_ref, wr_ref,
             rdeg_ref, y_ref, z2_ref):
        deg = d_ref[0] + d_ref[1]
        rdeg = 1.0 / jnp.maximum(deg, 1.0)
        h = jnp.maximum((a_ref[0] + a_ref[1]) * rdeg + z_ref[...] + bl_ref[...],
                        0.0)
        rdeg_ref[...] = rdeg
        y_ref[...] = jnp.dot(h, wl_ref[...], preferred_element_type=F32)
        z2_ref[...] = jnp.dot(h, wr_ref[...], preferred_element_type=F32)

    return pl.pallas_call(
        body,
        grid=(NP // BLK,),
        in_specs=[_pair_spec(H), _pair_spec(1), _row_spec(H),
                  _fixed_spec((1, H)), _fixed_spec((H, H)), _fixed_spec((H, H))],
        out_specs=[_row_spec(1), _row_spec(H), _row_spec(H)],
        out_shape=[jax.ShapeDtypeStruct((NP, 1), F32),
                   jax.ShapeDtypeStruct((NP, H), F32),
                   jax.ShapeDtypeStruct((NP, H), F32)],
    )(accp, degp, z, bl, wln, wrn)


def _tcmid(accp, rdeg, z, bl, wln, wrn):
    def body(a_ref, rd_ref, z_ref, bl_ref, wl_ref, wr_ref, y_ref, z2_ref):
        h = jnp.maximum((a_ref[0] + a_ref[1]) * rd_ref[...] + z_ref[...]
                        + bl_ref[...], 0.0)
        y_ref[...] = jnp.dot(h, wl_ref[...], preferred_element_type=F32)
        z2_ref[...] = jnp.dot(h, wr_ref[...], preferred_element_type=F32)

    return pl.pallas_call(
        body,
        grid=(NP // BLK,),
        in_specs=[_pair_spec(H), _row_spec(1), _row_spec(H),
                  _fixed_spec((1, H)), _fixed_spec((H, H)), _fixed_spec((H, H))],
        out_specs=[_row_spec(H), _row_spec(H)],
        out_shape=[jax.ShapeDtypeStruct((NP, H), F32)] * 2,
    )(accp, rdeg, z, bl, wln, wrn)


def _tc4(accp, rdeg, z, bl, whead):
    def body(a_ref, rd_ref, z_ref, bl_ref, wh_ref, p_ref):
        h = jnp.maximum((a_ref[0] + a_ref[1]) * rd_ref[...] + z_ref[...]
                        + bl_ref[...], 0.0)
        p_ref[...] = jnp.dot(h, wh_ref[...], preferred_element_type=F32)

    return pl.pallas_call(
        body,
        grid=(NP // BLK,),
        in_specs=[_pair_spec(H), _row_spec(1), _row_spec(H),
                  _fixed_spec((1, H)), _fixed_spec((H, 4))],
        out_specs=_row_spec(4),
        out_shape=jax.ShapeDtypeStruct((NP, 4), F32),
    )(accp, rdeg, z, bl, whead)


def _tc5(paccp, rdeg, p, b3, m3):
    def body(pa_ref, rd_ref, p_ref, b3_ref, m3_ref, out_ref):
        pr = (pa_ref[0] + pa_ref[1]) * rd_ref[...]
        out_ref[...] = p_ref[:, 1:4] + b3_ref[...] + pr * m3_ref[...]

    return pl.pallas_call(
        body,
        grid=(NP // BLK,),
        in_specs=[_pair_spec(1), _row_spec(1), _row_spec(4),
                  _fixed_spec((1, 3)), _fixed_spec((1, 3))],
        out_specs=_row_spec(3),
        out_shape=jax.ShapeDtypeStruct((NP, 3), F32),
    )(paccp, rdeg, p, b3, m3)


def kernel(x, edge_index, Wl0, bl0, Wr0, Wl1, bl1, Wr1, Wl2, bl2, Wr2,
           Wl3, bl3, Wr3, Wlp, blp, Wrp, Wdn, bdn, Wv, bv):
    row = edge_index[0]
    col = edge_index[1]
    pad = E_PAD - E
    rowp = jnp.concatenate(
        [row, jnp.full((pad,), N, jnp.int32)]).reshape(NW, CPW, CHUNK)
    colp = jnp.concatenate(
        [col, jnp.zeros((pad,), jnp.int32)]).reshape(NW, CPW, CHUNK)
    xp = jnp.pad(x, ((0, NP - N), (0, 0)))
    z2 = jnp.zeros((RPS, H), F32)
    z1 = jnp.zeros((RPS,), F32)
    on = jnp.ones((CHUNK,), F32)

    _seg64_deg, _seg64, _seg1 = _sc_kernels()

    y0, zz0 = _tc0(xp, Wl0, Wr0)
    acc0, deg0 = _seg64_deg(y0, colp, rowp, z2, z1, on)
    rdeg, y1, zz1 = _tc1(acc0, deg0.reshape(NC, NP, 1), zz0,
                         bl0.reshape(1, H), Wl1, Wr1)
    acc1 = _seg64(y1, colp, rowp, z2)
    y2, zz2 = _tcmid(acc1, rdeg, zz1, bl1.reshape(1, H), Wl2, Wr2)
    acc2 = _seg64(y2, colp, rowp, z2)
    y3, zz3 = _tcmid(acc2, rdeg, zz2, bl2.reshape(1, H), Wl3, Wr3)
    acc3 = _seg64(y3, colp, rowp, z2)
    whead = jnp.concatenate([Wlp, Wrp, Wdn, Wv], axis=1)
    p = _tc4(acc3, rdeg, zz3, bl3.reshape(1, H), whead)
    accp = _seg1(p[:, 0], colp, rowp, z1)
    b3 = jnp.stack([blp[0], bdn[0], bv[0]]).reshape(1, 3)
    m3 = jnp.array([[1.0, 0.0, 0.0]], F32)
    out = _tc5(accp.reshape(NC, NP, 1), rdeg, p, b3, m3)
    return out[:N]


# speedup vs baseline: 5.4458x; 1.0009x over previous
import functools

import jax
import jax.numpy as jnp
from jax import lax
from jax.experimental import pallas as pl
from jax.experimental.pallas import tpu as pltpu
from jax.experimental.pallas import tpu_sc as plsc

N = 10000
E = 320000
D_IN = 128
H = 64
NC = 2
NS = 16
NW = NC * NS
CHUNK = 128
CPW = 80
EPW = CPW * CHUNK
E_PAD = NW * EPW
NP = 10240
RPS = NP // NS
BLK = 1280
F32 = jnp.float32

NBUF = 8
NROUND = CPW // NBUF


def _ring(y, col_v, row_v, acc, gbuf, gsem, ssem, deg=None):
    for b in range(NBUF):
        pltpu.async_copy(y.at[col_v.at[b]], gbuf.at[b], gsem.at[b])

    def round_body(g, carry):
        base = g * NBUF
        for b in range(NBUF):
            j = base + b
            pltpu.make_async_copy(y.at[col_v.at[j]], gbuf.at[b],
                                  gsem.at[b]).wait()
            pltpu.async_copy(gbuf.at[b], acc.at[row_v.at[j]], ssem.at[b],
                             add=True)
            if deg is not None:
                ones_v, dacc, dsem = deg
                pltpu.async_copy(ones_v, dacc.at[row_v.at[j]], dsem.at[b],
                                 add=True)
        for b in range(NBUF):
            j = base + b
            jn = jnp.minimum(j + NBUF, CPW - 1)
            pltpu.make_async_copy(gbuf.at[b], acc.at[row_v.at[j]],
                                  ssem.at[b]).wait()
            if deg is not None:
                ones_v, dacc, dsem = deg
                pltpu.make_async_copy(ones_v, dacc.at[row_v.at[j]],
                                      dsem.at[b]).wait()
            pltpu.async_copy(y.at[col_v.at[jn]], gbuf.at[b], gsem.at[b])
        return carry

    lax.fori_loop(0, NROUND, round_body, 0)
    for b in range(NBUF):
        pltpu.make_async_copy(y.at[col_v.at[CPW - 1]], gbuf.at[b],
                              gsem.at[b]).wait()


@functools.lru_cache(maxsize=None)
def _sc_kernels():
    mesh = plsc.VectorSubcoreMesh(core_axis_name="c", subcore_axis_name="s",
                                  num_cores=NC, num_subcores=NS)

    @functools.partial(
        pl.kernel,
        out_type=[jax.ShapeDtypeStruct((NC, NP, H), F32),
                  jax.ShapeDtypeStruct((NC, NP), F32)],
        mesh=mesh,
        compiler_params=pltpu.CompilerParams(use_tc_tiling_on_sc=False),
        scratch_types=[
            pltpu.VMEM((CPW, CHUNK), jnp.int32),
            pltpu.VMEM((CPW, CHUNK), jnp.int32),
            pltpu.VMEM((NBUF, CHUNK, H), F32),
            pltpu.VMEM((CHUNK,), F32),
            pltpu.VMEM_SHARED((NP, H), F32),
            pltpu.VMEM_SHARED((NP,), F32),
            pltpu.SemaphoreType.DMA((NBUF,)),
            pltpu.SemaphoreType.DMA((NBUF,)),
            pltpu.SemaphoreType.DMA((NBUF,)),
        ],
    )
    def seg64_deg(y, colr, rowr, z2, z1, on, accout, degout,
                  col_v, row_v, gbuf, ones_v, acc, dacc, gsem, ssem, dsem):
        ci = lax.axis_index("c")
        si = lax.axis_index("s")
        wid = ci * NS + si
        pltpu.sync_copy(z2, acc.at[pl.ds(si * RPS, RPS)])
        pltpu.sync_copy(z1, dacc.at[pl.ds(si * RPS, RPS)])
        pltpu.sync_copy(on, ones_v)
        pltpu.sync_copy(colr.at[wid], col_v)
        pltpu.sync_copy(rowr.at[wid], row_v)
        plsc.subcore_barrier()
        _ring(y, col_v, row_v, acc, gbuf, gsem, ssem,
              deg=(ones_v, dacc, dsem))
        plsc.subcore_barrier()
        pltpu.sync_copy(acc.at[pl.ds(si * RPS, RPS)],
                        accout.at[ci, pl.ds(si * RPS, RPS)])
        pltpu.sync_copy(dacc.at[pl.ds(si * RPS, RPS)],
                        degout.at[ci, pl.ds(si * RPS, RPS)])

    @functools.partial(
        pl.kernel,
        out_type=jax.ShapeDtypeStruct((NC, NP, H), F32),
        mesh=mesh,
        compiler_params=pltpu.CompilerParams(use_tc_tiling_on_sc=False),
        scratch_types=[
            pltpu.VMEM((CPW, CHUNK), jnp.int32),
            pltpu.VMEM((CPW, CHUNK), jnp.int32),
            pltpu.VMEM((NBUF, CHUNK, H), F32),
            pltpu.VMEM_SHARED((NP, H), F32),
            pltpu.SemaphoreType.DMA((NBUF,)),
            pltpu.SemaphoreType.DMA((NBUF,)),
        ],
    )
    def seg64(y, colr, rowr, z2, accout, col_v, row_v, gbuf, acc, gsem, ssem):
        ci = lax.axis_index("c")
        si = lax.axis_index("s")
        wid = ci * NS + si
        pltpu.sync_copy(z2, acc.at[pl.ds(si * RPS, RPS)])
        pltpu.sync_copy(colr.at[wid], col_v)
        pltpu.sync_copy(rowr.at[wid], row_v)
        plsc.subcore_barrier()
        _ring(y, col_v, row_v, acc, gbuf, gsem, ssem)
        plsc.subcore_barrier()
        pltpu.sync_copy(acc.at[pl.ds(si * RPS, RPS)],
                        accout.at[ci, pl.ds(si * RPS, RPS)])

    @functools.partial(
        pl.kernel,
        out_type=jax.ShapeDtypeStruct((NC, NP), F32),
        mesh=mesh,
        compiler_params=pltpu.CompilerParams(use_tc_tiling_on_sc=False),
        scratch_types=[
            pltpu.VMEM((CPW, CHUNK), jnp.int32),
            pltpu.VMEM((CPW, CHUNK), jnp.int32),
            pltpu.VMEM((NBUF, CHUNK), F32),
            pltpu.VMEM_SHARED((NP,), F32),
            pltpu.SemaphoreType.DMA((NBUF,)),
            pltpu.SemaphoreType.DMA((NBUF,)),
        ],
    )
    def seg1(y, colr, rowr, z1, accout, col_v, row_v, gbuf, acc, gsem, ssem):
        ci = lax.axis_index("c")
        si = lax.axis_index("s")
        wid = ci * NS + si
        pltpu.sync_copy(z1, acc.at[pl.ds(si * RPS, RPS)])
        pltpu.sync_copy(colr.at[wid], col_v)
        pltpu.sync_copy(rowr.at[wid], row_v)
        plsc.subcore_barrier()
        _ring(y, col_v, row_v, acc, gbuf, gsem, ssem)
        plsc.subcore_barrier()
        pltpu.sync_copy(acc.at[pl.ds(si * RPS, RPS)],
                        accout.at[ci, pl.ds(si * RPS, RPS)])

    return seg64_deg, seg64, seg1


def _row_spec(width):
    return pl.BlockSpec((BLK, width), lambda i: (i, 0))


def _fixed_spec(shape):
    nd = len(shape)
    return pl.BlockSpec(shape, lambda i: (0,) * nd)


def _pair_spec(width):
    return pl.BlockSpec((NC, BLK, width), lambda i: (0, i, 0))


def _tc0(x, wl, wr):
    def body(x_ref, wl_ref, wr_ref, y_ref, z_ref):
        xb = x_ref[...]
        y_ref[...] = jnp.dot(xb, wl_ref[...], preferred_element_type=F32)
        z_ref[...] = jnp.dot(xb, wr_ref[...], preferred_element_type=F32)

    return pl.pallas_call(
        body,
        grid=(NP // BLK,),
        in_specs=[_row_spec(D_IN), _fixed_spec((D_IN, H)), _fixed_spec((D_IN, H))],
        out_specs=[_row_spec(H), _row_spec(H)],
        out_shape=[jax.ShapeDtypeStruct((NP, H), F32)] * 2,
    )(x, wl, wr)


def _tc1(accp, degp, z, bl, wln, wrn):
    def body(a_ref, d_ref, z_ref, bl_ref, wl_ref, wr_ref,
             rdeg_ref, y_ref, z2_ref):
        deg = d_ref[0] + d_ref[1]
        rdeg = 1.0 / jnp.maximum(deg, 1.0)
        h = jnp.maximum((a_ref[0] + a_ref[1]) * rdeg + z_ref[...] + bl_ref[...],
                        0.0)
        rdeg_ref[...] = rdeg
        y_ref[...] = jnp.dot(h, wl_ref[...], preferred_element_type=F32)
        z2_ref[...] = jnp.dot(h, wr_ref[...], preferred_element_type=F32)

    return pl.pallas_call(
        body,
        grid=(NP // BLK,),
        in_specs=[_pair_spec(H), _pair_spec(1), _row_spec(H),
                  _fixed_spec((1, H)), _fixed_spec((H, H)), _fixed_spec((H, H))],
        out_specs=[_row_spec(1), _row_spec(H), _row_spec(H)],
        out_shape=[jax.ShapeDtypeStruct((NP, 1), F32),
                   jax.ShapeDtypeStruct((NP, H), F32),
                   jax.ShapeDtypeStruct((NP, H), F32)],
    )(accp, degp, z, bl, wln, wrn)


def _tcmid(accp, rdeg, z, bl, wln, wrn):
    def body(a_ref, rd_ref, z_ref, bl_ref, wl_ref, wr_ref, y_ref, z2_ref):
        h = jnp.maximum((a_ref[0] + a_ref[1]) * rd_ref[...] + z_ref[...]
                        + bl_ref[...], 0.0)
        y_ref[...] = jnp.dot(h, wl_ref[...], preferred_element_type=F32)
        z2_ref[...] = jnp.dot(h, wr_ref[...], preferred_element_type=F32)

    return pl.pallas_call(
        body,
        grid=(NP // BLK,),
        in_specs=[_pair_spec(H), _row_spec(1), _row_spec(H),
                  _fixed_spec((1, H)), _fixed_spec((H, H)), _fixed_spec((H, H))],
        out_specs=[_row_spec(H), _row_spec(H)],
        out_shape=[jax.ShapeDtypeStruct((NP, H), F32)] * 2,
    )(accp, rdeg, z, bl, wln, wrn)


def _tc4(accp, rdeg, z, bl, whead):
    def body(a_ref, rd_ref, z_ref, bl_ref, wh_ref, p_ref):
        h = jnp.maximum((a_ref[0] + a_ref[1]) * rd_ref[...] + z_ref[...]
                        + bl_ref[...], 0.0)
        p_ref[...] = jnp.dot(h, wh_ref[...], preferred_element_type=F32)

    return pl.pallas_call(
        body,
        grid=(NP // BLK,),
        in_specs=[_pair_spec(H), _row_spec(1), _row_spec(H),
                  _fixed_spec((1, H)), _fixed_spec((H, 4))],
        out_specs=_row_spec(4),
        out_shape=jax.ShapeDtypeStruct((NP, 4), F32),
    )(accp, rdeg, z, bl, whead)


def _tc5(paccp, rdeg, p, b3, m3):
    def body(pa_ref, rd_ref, p_ref, b3_ref, m3_ref, out_ref):
        pr = (pa_ref[0] + pa_ref[1]) * rd_ref[...]
        out_ref[...] = p_ref[:, 1:4] + b3_ref[...] + pr * m3_ref[...]

    return pl.pallas_call(
        body,
        grid=(NP // BLK,),
        in_specs=[_pair_spec(1), _row_spec(1), _row_spec(4),
                  _fixed_spec((1, 3)), _fixed_spec((1, 3))],
        out_specs=_row_spec(3),
        out_shape=jax.ShapeDtypeStruct((NP, 3), F32),
    )(paccp, rdeg, p, b3, m3)


def kernel(x, edge_index, Wl0, bl0, Wr0, Wl1, bl1, Wr1, Wl2, bl2, Wr2,
           Wl3, bl3, Wr3, Wlp, blp, Wrp, Wdn, bdn, Wv, bv):
    row = edge_index[0]
    col = edge_index[1]
    pad = E_PAD - E
    pad_rows = N + (jnp.arange(pad, dtype=jnp.int32) % (NP - N))
    rowp = jnp.concatenate([row, pad_rows]).reshape(NW, CPW, CHUNK)
    colp = jnp.concatenate(
        [col, jnp.zeros((pad,), jnp.int32)]).reshape(NW, CPW, CHUNK)
    xp = jnp.pad(x, ((0, NP - N), (0, 0)))
    z2 = jnp.zeros((RPS, H), F32)
    z1 = jnp.zeros((RPS,), F32)
    on = jnp.ones((CHUNK,), F32)

    _seg64_deg, _seg64, _seg1 = _sc_kernels()

    y0, zz0 = _tc0(xp, Wl0, Wr0)
    acc0, deg0 = _seg64_deg(y0, colp, rowp, z2, z1, on)
    rdeg, y1, zz1 = _tc1(acc0, deg0.reshape(NC, NP, 1), zz0,
                         bl0.reshape(1, H), Wl1, Wr1)
    acc1 = _seg64(y1, colp, rowp, z2)
    y2, zz2 = _tcmid(acc1, rdeg, zz1, bl1.reshape(1, H), Wl2, Wr2)
    acc2 = _seg64(y2, colp, rowp, z2)
    y3, zz3 = _tcmid(acc2, rdeg, zz2, bl2.reshape(1, H), Wl3, Wr3)
    acc3 = _seg64(y3, colp, rowp, z2)
    whead = jnp.concatenate([Wlp, Wrp, Wdn, Wv], axis=1)
    p = _tc4(acc3, rdeg, zz3, bl3.reshape(1, H), whead)
    accp = _seg1(p[:, 0], colp, rowp, z1)
    b3 = jnp.stack([blp[0], bdn[0], bv[0]]).reshape(1, 3)
    m3 = jnp.array([[1.0, 0.0, 0.0]], F32)
    out = _tc5(accp.reshape(NC, NP, 1), rdeg, p, b3, m3)
    return out[:N]


# speedup vs baseline: 5.6936x; 1.0455x over previous
import functools

import jax
import jax.numpy as jnp
from jax import lax
from jax.experimental import pallas as pl
from jax.experimental.pallas import tpu as pltpu
from jax.experimental.pallas import tpu_sc as plsc

N = 10000
E = 320000
D_IN = 128
H = 64
NC = 2
NS = 16
NW = NC * NS
CHUNK = 128
CPW = 80
E_PAD = NW * CPW * CHUNK
NP = 10240
RPS = NP // NS
BLK = 1280
F32 = jnp.float32

NBUF = 4
NROUND = CPW // NBUF
NCOPY = RPS // CHUNK


def _fill2d(buf, rows, value):
    v = jnp.full((16,), value, F32)

    def body(r, c):
        for k in range(H // 16):
            buf[r, pl.ds(k * 16, 16)] = v
        return c

    lax.fori_loop(0, rows, body, 0)


def _fill1d(buf, n, value):
    v = jnp.full((16,), value, F32)

    def body(i, c):
        buf[pl.ds(i * 16, 16)] = v
        return c

    lax.fori_loop(0, n // 16, body, 0)


def _ring(y, col_v, row_v, acc, gbuf, gsem, ssem, deg=None):
    for b in range(NBUF):
        pltpu.async_copy(y.at[col_v.at[b]], gbuf.at[b], gsem.at[b])

    def round_body(g, carry):
        base = g * NBUF
        for b in range(NBUF):
            j = base + b
            pltpu.make_async_copy(y.at[col_v.at[j]], gbuf.at[b],
                                  gsem.at[b]).wait()
            pltpu.async_copy(gbuf.at[b], acc.at[row_v.at[j]], ssem.at[b],
                             add=True)
            if deg is not None:
                ones_v, dacc, dsem = deg
                pltpu.async_copy(ones_v, dacc.at[row_v.at[j]], dsem.at[b],
                                 add=True)
        for b in range(NBUF):
            j = base + b
            jn = jnp.minimum(j + NBUF, CPW - 1)
            pltpu.make_async_copy(gbuf.at[b], acc.at[row_v.at[j]],
                                  ssem.at[b]).wait()
            if deg is not None:
                ones_v, dacc, dsem = deg
                pltpu.make_async_copy(ones_v, dacc.at[row_v.at[j]],
                                      dsem.at[b]).wait()
            pltpu.async_copy(y.at[col_v.at[jn]], gbuf.at[b], gsem.at[b])
        return carry

    lax.fori_loop(0, NROUND, round_body, 0)
    for b in range(NBUF):
        pltpu.make_async_copy(y.at[col_v.at[CPW - 1]], gbuf.at[b],
                              gsem.at[b]).wait()


def _copy_out_2d(acc, accout, ci, si, csem):
    for t in range(NCOPY):
        pltpu.async_copy(acc.at[pl.ds(si * RPS + t * CHUNK, CHUNK)],
                         accout.at[ci, pl.ds(si * RPS + t * CHUNK, CHUNK)],
                         csem.at[t])
    for t in range(NCOPY):
        pltpu.make_async_copy(
            acc.at[pl.ds(si * RPS + t * CHUNK, CHUNK)],
            accout.at[ci, pl.ds(si * RPS + t * CHUNK, CHUNK)],
            csem.at[t]).wait()


@functools.lru_cache(maxsize=None)
def _sc_kernels():
    mesh = plsc.VectorSubcoreMesh(core_axis_name="c", subcore_axis_name="s",
                                  num_cores=NC, num_subcores=NS)

    @functools.partial(
        pl.kernel,
        out_type=[jax.ShapeDtypeStruct((NC, NP, H), F32),
                  jax.ShapeDtypeStruct((NC, NP), F32)],
        mesh=mesh,
        compiler_params=pltpu.CompilerParams(use_tc_tiling_on_sc=False),
        scratch_types=[
            pltpu.VMEM((CPW, CHUNK), jnp.int32),
            pltpu.VMEM((CPW, CHUNK), jnp.int32),
            pltpu.VMEM((NBUF, CHUNK, H), F32),
            pltpu.VMEM((CHUNK,), F32),
            pltpu.VMEM((RPS,), F32),
            pltpu.VMEM_SHARED((NP, H), F32),
            pltpu.VMEM_SHARED((NP,), F32),
            pltpu.SemaphoreType.DMA((NBUF,)),
            pltpu.SemaphoreType.DMA((NBUF,)),
            pltpu.SemaphoreType.DMA((NBUF,)),
            pltpu.SemaphoreType.DMA((NCOPY,)),
        ],
    )
    def seg64_deg(y, colr, rowr, accout, degout,
                  col_v, row_v, gbuf, ones_v, zbuf1, acc, dacc,
                  gsem, ssem, dsem, csem):
        ci = lax.axis_index("c")
        si = lax.axis_index("s")
        wid = ci * NS + si
        zbuf = gbuf.at[0]
        _fill2d(zbuf, CHUNK, 0.0)
        _fill1d(zbuf1, RPS, 0.0)
        _fill1d(ones_v, CHUNK, 1.0)
        for t in range(RPS // CHUNK):
            pltpu.sync_copy(zbuf, acc.at[pl.ds(si * RPS + t * CHUNK, CHUNK)])
        pltpu.sync_copy(zbuf1, dacc.at[pl.ds(si * RPS, RPS)])
        pltpu.sync_copy(colr.at[wid], col_v)
        pltpu.sync_copy(rowr.at[wid], row_v)
        plsc.subcore_barrier()
        _ring(y, col_v, row_v, acc, gbuf, gsem, ssem,
              deg=(ones_v, dacc, dsem))
        plsc.subcore_barrier()
        _copy_out_2d(acc, accout, ci, si, csem)
        pltpu.sync_copy(dacc.at[pl.ds(si * RPS, RPS)],
                        degout.at[ci, pl.ds(si * RPS, RPS)])

    @functools.partial(
        pl.kernel,
        out_type=jax.ShapeDtypeStruct((NC, NP, H), F32),
        mesh=mesh,
        compiler_params=pltpu.CompilerParams(use_tc_tiling_on_sc=False),
        scratch_types=[
            pltpu.VMEM((CPW, CHUNK), jnp.int32),
            pltpu.VMEM((CPW, CHUNK), jnp.int32),
            pltpu.VMEM((NBUF, CHUNK, H), F32),
            pltpu.VMEM_SHARED((NP, H), F32),
            pltpu.SemaphoreType.DMA((NBUF,)),
            pltpu.SemaphoreType.DMA((NBUF,)),
            pltpu.SemaphoreType.DMA((NCOPY,)),
        ],
    )
    def seg64(y, colr, rowr, accout, col_v, row_v, gbuf, acc,
              gsem, ssem, csem):
        ci = lax.axis_index("c")
        si = lax.axis_index("s")
        wid = ci * NS + si
        zbuf = gbuf.at[0]
        _fill2d(zbuf, CHUNK, 0.0)
        for t in range(RPS // CHUNK):
            pltpu.sync_copy(zbuf, acc.at[pl.ds(si * RPS + t * CHUNK, CHUNK)])
        pltpu.sync_copy(colr.at[wid], col_v)
        pltpu.sync_copy(rowr.at[wid], row_v)
        plsc.subcore_barrier()
        _ring(y, col_v, row_v, acc, gbuf, gsem, ssem)
        plsc.subcore_barrier()
        _copy_out_2d(acc, accout, ci, si, csem)

    @functools.partial(
        pl.kernel,
        out_type=jax.ShapeDtypeStruct((NC, NP), F32),
        mesh=mesh,
        compiler_params=pltpu.CompilerParams(use_tc_tiling_on_sc=False),
        scratch_types=[
            pltpu.VMEM((CPW, CHUNK), jnp.int32),
            pltpu.VMEM((CPW, CHUNK), jnp.int32),
            pltpu.VMEM((NBUF, CHUNK), F32),
            pltpu.VMEM((RPS,), F32),
            pltpu.VMEM_SHARED((NP,), F32),
            pltpu.SemaphoreType.DMA((NBUF,)),
            pltpu.SemaphoreType.DMA((NBUF,)),
        ],
    )
    def seg1(y, colr, rowr, accout, col_v, row_v, gbuf, zbuf1, acc,
             gsem, ssem):
        ci = lax.axis_index("c")
        si = lax.axis_index("s")
        wid = ci * NS + si
        _fill1d(zbuf1, RPS, 0.0)
        pltpu.sync_copy(zbuf1, acc.at[pl.ds(si * RPS, RPS)])
        pltpu.sync_copy(colr.at[wid], col_v)
        pltpu.sync_copy(rowr.at[wid], row_v)
        plsc.subcore_barrier()
        _ring(y, col_v, row_v, acc, gbuf, gsem, ssem)
        plsc.subcore_barrier()
        pltpu.sync_copy(acc.at[pl.ds(si * RPS, RPS)],
                        accout.at[ci, pl.ds(si * RPS, RPS)])

    return seg64_deg, seg64, seg1


def _row_spec(width):
    return pl.BlockSpec((BLK, width), lambda i: (i, 0))


def _fixed_spec(shape):
    nd = len(shape)
    return pl.BlockSpec(shape, lambda i: (0,) * nd)


def _pair_spec(width):
    return pl.BlockSpec((NC, BLK, width), lambda i: (0, i, 0))


def _tc0(x, wl, wr):
    def body(x_ref, wl_ref, wr_ref, y_ref, z_ref):
        xb = x_ref[...]
        y_ref[...] = jnp.dot(xb, wl_ref[...], preferred_element_type=F32)
        z_ref[...] = jnp.dot(xb, wr_ref[...], preferred_element_type=F32)

    return pl.pallas_call(
        body,
        grid=(NP // BLK,),
        in_specs=[_row_spec(D_IN), _fixed_spec((D_IN, H)), _fixed_spec((D_IN, H))],
        out_specs=[_row_spec(H), _row_spec(H)],
        out_shape=[jax.ShapeDtypeStruct((NP, H), F32)] * 2,
    )(x, wl, wr)


def _tc1(accp, degp, z, bl, wln, wrn):
    def body(a_ref, d_ref, z_ref, bl_ref, wl_ref, wr_ref,
             rdeg_ref, y_ref, z2_ref):
        deg = d_ref[0] + d_ref[1]
        rdeg = 1.0 / jnp.maximum(deg, 1.0)
        h = jnp.maximum((a_ref[0] + a_ref[1]) * rdeg + z_ref[...] + bl_ref[...],
                        0.0)
        rdeg_ref[...] = rdeg
        y_ref[...] = jnp.dot(h, wl_ref[...], preferred_element_type=F32)
        z2_ref[...] = jnp.dot(h, wr_ref[...], preferred_element_type=F32)

    return pl.pallas_call(
        body,
        grid=(NP // BLK,),
        in_specs=[_pair_spec(H), _pair_spec(1), _row_spec(H),
                  _fixed_spec((1, H)), _fixed_spec((H, H)), _fixed_spec((H, H))],
        out_specs=[_row_spec(1), _row_spec(H), _row_spec(H)],
        out_shape=[jax.ShapeDtypeStruct((NP, 1), F32),
                   jax.ShapeDtypeStruct((NP, H), F32),
                   jax.ShapeDtypeStruct((NP, H), F32)],
    )(accp, degp, z, bl, wln, wrn)


def _tcmid(accp, rdeg, z, bl, wln, wrn):
    def body(a_ref, rd_ref, z_ref, bl_ref, wl_ref, wr_ref, y_ref, z2_ref):
        h = jnp.maximum((a_ref[0] + a_ref[1]) * rd_ref[...] + z_ref[...]
                        + bl_ref[...], 0.0)
        y_ref[...] = jnp.dot(h, wl_ref[...], preferred_element_type=F32)
        z2_ref[...] = jnp.dot(h, wr_ref[...], preferred_element_type=F32)

    return pl.pallas_call(
        body,
        grid=(NP // BLK,),
        in_specs=[_pair_spec(H), _row_spec(1), _row_spec(H),
                  _fixed_spec((1, H)), _fixed_spec((H, H)), _fixed_spec((H, H))],
        out_specs=[_row_spec(H), _row_spec(H)],
        out_shape=[jax.ShapeDtypeStruct((NP, H), F32)] * 2,
    )(accp, rdeg, z, bl, wln, wrn)


def _tc4(accp, rdeg, z, bl, whead):
    def body(a_ref, rd_ref, z_ref, bl_ref, wh_ref, p_ref):
        h = jnp.maximum((a_ref[0] + a_ref[1]) * rd_ref[...] + z_ref[...]
                        + bl_ref[...], 0.0)
        p_ref[...] = jnp.dot(h, wh_ref[...], preferred_element_type=F32)

    return pl.pallas_call(
        body,
        grid=(NP // BLK,),
        in_specs=[_pair_spec(H), _row_spec(1), _row_spec(H),
                  _fixed_spec((1, H)), _fixed_spec((H, 4))],
        out_specs=_row_spec(4),
        out_shape=jax.ShapeDtypeStruct((NP, 4), F32),
    )(accp, rdeg, z, bl, whead)


def _tc5(paccp, rdeg, p, b3, m3):
    def body(pa_ref, rd_ref, p_ref, b3_ref, m3_ref, out_ref):
        pr = (pa_ref[0] + pa_ref[1]) * rd_ref[...]
        out_ref[...] = p_ref[:, 1:4] + b3_ref[...] + pr * m3_ref[...]

    return pl.pallas_call(
        body,
        grid=(NP // BLK,),
        in_specs=[_pair_spec(1), _row_spec(1), _row_spec(4),
                  _fixed_spec((1, 3)), _fixed_spec((1, 3))],
        out_specs=_row_spec(3),
        out_shape=jax.ShapeDtypeStruct((NP, 3), F32),
    )(paccp, rdeg, p, b3, m3)


def kernel(x, edge_index, Wl0, bl0, Wr0, Wl1, bl1, Wr1, Wl2, bl2, Wr2,
           Wl3, bl3, Wr3, Wlp, blp, Wrp, Wdn, bdn, Wv, bv):
    row = edge_index[0]
    col = edge_index[1]
    ext = E_PAD - E
    pad_rows = N + (jnp.arange(ext, dtype=jnp.int32) % (NP - N))
    rowp = jnp.concatenate([row, pad_rows]).reshape(NW, CPW, CHUNK)
    colp = jnp.concatenate(
        [col, jnp.zeros((ext,), jnp.int32)]).reshape(NW, CPW, CHUNK)
    xp = jnp.pad(x, ((0, NP - N), (0, 0)))

    _seg64_deg, _seg64, _seg1 = _sc_kernels()

    y0, zz0 = _tc0(xp, Wl0, Wr0)
    acc0, deg0 = _seg64_deg(y0, colp, rowp)
    rdeg, y1, zz1 = _tc1(acc0, deg0.reshape(NC, NP, 1), zz0,
                         bl0.reshape(1, H), Wl1, Wr1)
    acc1 = _seg64(y1, colp, rowp)
    y2, zz2 = _tcmid(acc1, rdeg, zz1, bl1.reshape(1, H), Wl2, Wr2)
    acc2 = _seg64(y2, colp, rowp)
    y3, zz3 = _tcmid(acc2, rdeg, zz2, bl2.reshape(1, H), Wl3, Wr3)
    acc3 = _seg64(y3, colp, rowp)
    whead = jnp.concatenate([Wlp, Wrp, Wdn, Wv], axis=1)
    p = _tc4(acc3, rdeg, zz3, bl3.reshape(1, H), whead)
    accp = _seg1(p[:, 0], colp, rowp)
    b3 = jnp.stack([blp[0], bdn[0], bv[0]]).reshape(1, 3)
    m3 = jnp.array([[1.0, 0.0, 0.0]], F32)
    out = _tc5(accp.reshape(NC, NP, 1), rdeg, p, b3, m3)
    return out[:N]
